# baseline (speedup 1.0000x reference)
import functools

import jax
import jax.numpy as jnp
from jax import lax
from jax.experimental import pallas as pl
from jax.experimental.pallas import tpu as pltpu
from jax.experimental.pallas import tpu_sc as plsc

H = 128
LANES = 16
NJ = H // LANES
EPS = 1e-12
CHUNK = 80
NSLOT = 4


def _rsqrt_scalar(x):
    i = lax.bitcast_convert_type(x, jnp.int32)
    i = jnp.int32(0x5F3759DF) - (i >> 1)
    y = lax.bitcast_convert_type(i, jnp.float32)
    for _ in range(3):
        y = y * (1.5 - 0.5 * x * y * y)
    return y


def _make_sc_call(n_rows, v, pt_rows):
    info = plsc.get_sparse_core_info()
    nw = info.num_cores * info.num_subcores
    rows_per_w = n_rows // nw
    n_chunks = rows_per_w // CHUNK
    assert pt_rows & (pt_rows - 1) == 0, "position count must be a power of 2"
    p_shift = (pt_rows - 1).bit_length()
    mesh = plsc.VectorSubcoreMesh(core_axis_name="c", subcore_axis_name="s")

    @functools.partial(
        pl.kernel,
        out_type=jax.ShapeDtypeStruct((n_rows, H), jnp.float32),
        mesh=mesh,
        scratch_types=[
            pltpu.VMEM((NSLOT, 2, CHUNK), jnp.int32),
            pltpu.VMEM((NSLOT, CHUNK, H), jnp.float32),
            pltpu.VMEM((H,), jnp.float32),
            pltpu.VMEM((H,), jnp.float32),
            pltpu.VMEM((H,), jnp.float32),
            pltpu.VMEM((pt_rows, H), jnp.float32),
            pltpu.SemaphoreType.DMA,
            pltpu.SemaphoreType.DMA,
            pltpu.SemaphoreType.DMA,
            pltpu.SemaphoreType.DMA,
        ],
        compiler_params=pltpu.CompilerParams(needs_layout_passes=False),
    )
    def sc_call(idx2_h, word_h, ptab_h, tdiff_h, gam_h, bet_h, out_h,
                idx_v, gbuf_v, gam_v, bet_v, tdiff_v, ptab_s,
                wsem, osem0, osem1, isem):
        wid = lax.axis_index("s") * info.num_cores + lax.axis_index("c")
        base_w = wid * rows_per_w

        pltpu.sync_copy(gam_h, gam_v)
        pltpu.sync_copy(bet_h, bet_v)
        pltpu.sync_copy(tdiff_h, tdiff_v)
        pltpu.sync_copy(ptab_h, ptab_s)

        def idx_handle(c):
            return pltpu.make_async_copy(
                idx2_h.at[wid * n_chunks + c],
                idx_v.at[c % NSLOT], isem)

        def wg_handle(c):
            s = c % NSLOT
            return pltpu.make_async_copy(word_h.at[idx_v.at[s, 0]],
                                         gbuf_v.at[s], wsem)

        def out_handle(c, sem):
            return pltpu.make_async_copy(
                gbuf_v.at[c % NSLOT],
                out_h.at[pl.ds(base_w + c * CHUNK, CHUNK)], sem)

        pltpu.sync_copy(idx2_h.at[wid * n_chunks], idx_v.at[0])
        wg_handle(0).start()
        pltpu.sync_copy(idx2_h.at[wid * n_chunks + 1], idx_v.at[1])
        wg_handle(1).start()
        idx_handle(2).start()

        def chunk_body(c, carry):
            s = c % NSLOT
            even = (c % 2) == 0
            wg_handle(c).wait()

            @pl.when(jnp.logical_and(c > 3, even))
            def _():
                out_handle(c - 4, osem0).wait()

            @pl.when(jnp.logical_and(c > 3, jnp.logical_not(even)))
            def _():
                out_handle(c - 4, osem1).wait()

            @pl.when(c + 2 < n_chunks)
            def _():
                idx_handle(c + 2).wait()
                wg_handle(c + 2).start()

            @pl.when(c + 3 < n_chunks)
            def _():
                idx_handle(c + 3).start()

            def group_body(gi, rcarry):
                ptv = idx_v[s, 1, pl.ds(gi * LANES, LANES)]
                gs = [gam_v[pl.ds(j * LANES, LANES)] for j in range(NJ)]
                bs = [bet_v[pl.ds(j * LANES, LANES)] for j in range(NJ)]
                ds_ = [tdiff_v[pl.ds(j * LANES, LANES)] for j in range(NJ)]
                for r16 in range(LANES):
                    r = gi * LANES + r16
                    ptid = ptv[r16]
                    pid = ptid & jnp.int32(pt_rows - 1)
                    tf = (ptid >> p_shift).astype(jnp.float32)
                    xs = [gbuf_v[s, r, pl.ds(j * LANES, LANES)]
                          + ptab_s[pid, pl.ds(j * LANES, LANES)]
                          + tf * ds_[j]
                          for j in range(NJ)]
                    sums = xs
                    sqs = [x * x for x in xs]
                    while len(sums) > 1:
                        sums = [a + b for a, b in zip(sums[::2], sums[1::2])]
                        sqs = [a + b for a, b in zip(sqs[::2], sqs[1::2])]
                    rs = jnp.sum(sums[0])
                    rq = jnp.sum(sqs[0])
                    mean = rs * (1.0 / H)
                    var = jnp.maximum(rq * (1.0 / H) - mean * mean, 0.0)
                    inv = _rsqrt_scalar(var + EPS)
                    for j in range(NJ):
                        gbuf_v[s, r, pl.ds(j * LANES, LANES)] = (
                            (xs[j] - mean) * inv) * gs[j] + bs[j]
                return rcarry

            lax.fori_loop(0, CHUNK // LANES, group_body, 0, unroll=False)

            @pl.when(even)
            def _():
                out_handle(c, osem0).start()

            @pl.when(jnp.logical_not(even))
            def _():
                out_handle(c, osem1).start()

            return carry

        lax.fori_loop(0, n_chunks, chunk_body, 0, unroll=False)
        out_handle(n_chunks - 4, osem0).wait()
        out_handle(n_chunks - 3, osem1).wait()
        out_handle(n_chunks - 2, osem0).wait()
        out_handle(n_chunks - 1, osem1).wait()

    return sc_call


def kernel(input_ids, position_ids, token_type_ids, word_embeddings,
           position_embeddings, token_type_embeddings, ln_gamma, ln_beta):
    b, l = input_ids.shape
    v, h = word_embeddings.shape
    p = position_embeddings.shape[0]
    t = token_type_embeddings.shape[0]
    n_rows = b * l
    ids = input_ids.reshape(n_rows).astype(jnp.int32)
    ptids = (token_type_ids.reshape(n_rows).astype(jnp.int32) * p
             + position_ids.reshape(n_rows).astype(jnp.int32))
    info = plsc.get_sparse_core_info()
    nw = info.num_cores * info.num_subcores
    n_chunks = n_rows // (nw * CHUNK)
    idx2 = (jnp.stack([ids, ptids])
            .reshape(2, nw * n_chunks, CHUNK)
            .transpose(1, 0, 2))
    ptab0 = position_embeddings + token_type_embeddings[0][None, :]
    tdiff = token_type_embeddings[1] - token_type_embeddings[0]
    sc_call = _make_sc_call(n_rows, v, p)
    out = sc_call(idx2, word_embeddings, ptab0, tdiff, ln_gamma, ln_beta)
    return out.reshape(b, l, h)


# speedup vs baseline: 1.1768x; 1.1768x over previous
import functools

import jax
import jax.numpy as jnp
from jax import lax
from jax.experimental import pallas as pl
from jax.experimental.pallas import tpu as pltpu
from jax.experimental.pallas import tpu_sc as plsc

H = 128
LANES = 16
NJ = H // LANES
EPS = 1e-12
CHUNK = 128
NSLOT = 6


def _rsqrt_scalar(x):
    i = lax.bitcast_convert_type(x, jnp.int32)
    i = jnp.int32(0x5F3759DF) - (i >> 1)
    y = lax.bitcast_convert_type(i, jnp.float32)
    for _ in range(3):
        y = y * (1.5 - 0.5 * x * y * y)
    return y


def _make_sc_call(n_rows, v, pt_rows):
    info = plsc.get_sparse_core_info()
    nw = info.num_cores * info.num_subcores
    rows_per_w = n_rows // nw
    n_chunks = rows_per_w // CHUNK
    mesh = plsc.VectorSubcoreMesh(core_axis_name="c", subcore_axis_name="s")

    @functools.partial(
        pl.kernel,
        out_type=jax.ShapeDtypeStruct((n_rows, H), jnp.float32),
        mesh=mesh,
        scratch_types=[
            pltpu.VMEM((NSLOT, 2, CHUNK), jnp.int32),
            pltpu.VMEM((NSLOT, CHUNK, H), jnp.float32),
            pltpu.VMEM((H,), jnp.float32),
            pltpu.VMEM((H,), jnp.float32),
            pltpu.SemaphoreType.DMA,
            pltpu.SemaphoreType.DMA,
            pltpu.SemaphoreType.DMA,
            pltpu.SemaphoreType.DMA,
            pltpu.SemaphoreType.DMA,
        ],
        compiler_params=pltpu.CompilerParams(needs_layout_passes=False),
    )
    def sc_call(idx2_h, word_h, ptab_h, gam_h, bet_h, out_h,
                idx_v, gbuf_v, gam_v, bet_v,
                wsem, psem, osem0, osem1, isem):
        wid = lax.axis_index("s") * info.num_cores + lax.axis_index("c")
        base_w = wid * rows_per_w

        pltpu.sync_copy(gam_h, gam_v)
        pltpu.sync_copy(bet_h, bet_v)

        def idx_handle(c):
            return pltpu.make_async_copy(
                idx2_h.at[:, pl.ds(base_w + c * CHUNK, CHUNK)],
                idx_v.at[c % NSLOT], isem)

        def wg_handle(c):
            s = c % NSLOT
            return pltpu.make_async_copy(word_h.at[idx_v.at[s, 0]],
                                         gbuf_v.at[s], wsem)

        def pgather(c):
            s = c % NSLOT
            pltpu.async_copy(ptab_h.at[idx_v.at[s, 1]], gbuf_v.at[s],
                             psem, add=True)

        def wait_pgather(c):
            s = c % NSLOT
            pltpu.make_async_copy(ptab_h.at[idx_v.at[s, 1]], gbuf_v.at[s],
                                  psem).wait()

        def out_handle(c, sem):
            return pltpu.make_async_copy(
                gbuf_v.at[c % NSLOT],
                out_h.at[pl.ds(base_w + c * CHUNK, CHUNK)], sem)

        pltpu.sync_copy(idx2_h.at[:, pl.ds(base_w, CHUNK)], idx_v.at[0])
        wg_handle(0).start()
        wg_handle(0).wait()
        pgather(0)
        pltpu.sync_copy(idx2_h.at[:, pl.ds(base_w + CHUNK, CHUNK)],
                        idx_v.at[1])
        wg_handle(1).start()
        idx_handle(2).start()

        def chunk_body(c, carry):
            s = c % NSLOT
            even = (c % 2) == 0
            wait_pgather(c)

            @pl.when(c + 1 < n_chunks)
            def _():
                wg_handle(c + 1).wait()
                pgather(c + 1)

            @pl.when(jnp.logical_and(c > 3, even))
            def _():
                out_handle(c - 4, osem0).wait()

            @pl.when(jnp.logical_and(c > 3, jnp.logical_not(even)))
            def _():
                out_handle(c - 4, osem1).wait()

            @pl.when(c + 2 < n_chunks)
            def _():
                idx_handle(c + 2).wait()
                wg_handle(c + 2).start()

            @pl.when(c + 3 < n_chunks)
            def _():
                idx_handle(c + 3).start()

            def group_body(gi, rcarry):
                gs = [gam_v[pl.ds(j * LANES, LANES)] for j in range(NJ)]
                bs = [bet_v[pl.ds(j * LANES, LANES)] for j in range(NJ)]
                for r16 in range(LANES):
                    r = gi * LANES + r16
                    xs = [gbuf_v[s, r, pl.ds(j * LANES, LANES)]
                          for j in range(NJ)]
                    sums = xs
                    sqs = [x * x for x in xs]
                    while len(sums) > 1:
                        sums = [a + b for a, b in zip(sums[::2], sums[1::2])]
                        sqs = [a + b for a, b in zip(sqs[::2], sqs[1::2])]
                    rs = jnp.sum(sums[0])
                    rq = jnp.sum(sqs[0])
                    mean = rs * (1.0 / H)
                    var = jnp.maximum(rq * (1.0 / H) - mean * mean, 0.0)
                    inv = _rsqrt_scalar(var + EPS)
                    for j in range(NJ):
                        gbuf_v[s, r, pl.ds(j * LANES, LANES)] = (
                            (xs[j] - mean) * inv) * gs[j] + bs[j]
                return rcarry

            lax.fori_loop(0, CHUNK // LANES, group_body, 0, unroll=False)

            @pl.when(even)
            def _():
                out_handle(c, osem0).start()

            @pl.when(jnp.logical_not(even))
            def _():
                out_handle(c, osem1).start()

            return carry

        lax.fori_loop(0, n_chunks, chunk_body, 0, unroll=False)
        out_handle(n_chunks - 4, osem0).wait()
        out_handle(n_chunks - 3, osem1).wait()
        out_handle(n_chunks - 2, osem0).wait()
        out_handle(n_chunks - 1, osem1).wait()

    return sc_call


def kernel(input_ids, position_ids, token_type_ids, word_embeddings,
           position_embeddings, token_type_embeddings, ln_gamma, ln_beta):
    b, l = input_ids.shape
    v, h = word_embeddings.shape
    p = position_embeddings.shape[0]
    t = token_type_embeddings.shape[0]
    n_rows = b * l
    ids = input_ids.reshape(n_rows).astype(jnp.int32)
    ptids = (token_type_ids.reshape(n_rows).astype(jnp.int32) * p
             + position_ids.reshape(n_rows).astype(jnp.int32))
    idx2 = jnp.stack([ids, ptids])
    ptab = (position_embeddings[None, :, :]
            + token_type_embeddings[:, None, :]).reshape(t * p, h)
    sc_call = _make_sc_call(n_rows, v, t * p)
    out = sc_call(idx2, word_embeddings, ptab, ln_gamma, ln_beta)
    return out.reshape(b, l, h)


# speedup vs baseline: 1.5285x; 1.2989x over previous
import functools

import jax
import jax.numpy as jnp
from jax import lax
from jax.experimental import pallas as pl
from jax.experimental.pallas import tpu as pltpu
from jax.experimental.pallas import tpu_sc as plsc

H = 128
LANES = 16
NJ = H // LANES
EPS = 1e-12
CHUNK = 128
NSLOT = 6


def _rsqrt_scalar(x):
    i = lax.bitcast_convert_type(x, jnp.int32)
    i = jnp.int32(0x5F3759DF) - (i >> 1)
    y = lax.bitcast_convert_type(i, jnp.float32)
    for _ in range(2):
        y = y * (1.5 - 0.5 * x * y * y)
    return y


def _make_sc_call(n_rows, v, pt_rows):
    info = plsc.get_sparse_core_info()
    nw = info.num_cores * info.num_subcores
    rows_per_w = n_rows // nw
    n_chunks = rows_per_w // CHUNK
    mesh = plsc.VectorSubcoreMesh(core_axis_name="c", subcore_axis_name="s")

    @functools.partial(
        pl.kernel,
        out_type=jax.ShapeDtypeStruct((n_rows, H), jnp.float32),
        mesh=mesh,
        scratch_types=[
            pltpu.VMEM((NSLOT, 2, CHUNK), jnp.int32),
            pltpu.VMEM((NSLOT, CHUNK, H), jnp.float32),
            pltpu.VMEM((H,), jnp.float32),
            pltpu.VMEM((H,), jnp.float32),
            pltpu.SemaphoreType.DMA,
            pltpu.SemaphoreType.DMA,
            pltpu.SemaphoreType.DMA,
            pltpu.SemaphoreType.DMA,
            pltpu.SemaphoreType.DMA,
        ],
        compiler_params=pltpu.CompilerParams(needs_layout_passes=False),
    )
    def sc_call(idx2_h, word_h, ptab_h, gam_h, bet_h, out_h,
                idx_v, gbuf_v, gam_v, bet_v,
                wsem, psem, osem0, osem1, isem):
        wid = lax.axis_index("s") * info.num_cores + lax.axis_index("c")
        base_w = wid * rows_per_w

        pltpu.sync_copy(gam_h, gam_v)
        pltpu.sync_copy(bet_h, bet_v)

        def idx_handle(c):
            return pltpu.make_async_copy(
                idx2_h.at[:, pl.ds(base_w + c * CHUNK, CHUNK)],
                idx_v.at[c % NSLOT], isem)

        def wg_handle(c):
            s = c % NSLOT
            return pltpu.make_async_copy(word_h.at[idx_v.at[s, 0]],
                                         gbuf_v.at[s], wsem)

        def pgather(c):
            s = c % NSLOT
            pltpu.async_copy(ptab_h.at[idx_v.at[s, 1]], gbuf_v.at[s],
                             psem, add=True)

        def wait_pgather(c):
            s = c % NSLOT
            pltpu.make_async_copy(ptab_h.at[idx_v.at[s, 1]], gbuf_v.at[s],
                                  psem).wait()

        def out_handle(c, sem):
            return pltpu.make_async_copy(
                gbuf_v.at[c % NSLOT],
                out_h.at[pl.ds(base_w + c * CHUNK, CHUNK)], sem)

        pltpu.sync_copy(idx2_h.at[:, pl.ds(base_w, CHUNK)], idx_v.at[0])
        wg_handle(0).start()
        wg_handle(0).wait()
        pgather(0)
        pltpu.sync_copy(idx2_h.at[:, pl.ds(base_w + CHUNK, CHUNK)],
                        idx_v.at[1])
        wg_handle(1).start()
        idx_handle(2).start()

        def chunk_body(c, carry):
            s = c % NSLOT
            even = (c % 2) == 0
            wait_pgather(c)

            @pl.when(c + 1 < n_chunks)
            def _():
                wg_handle(c + 1).wait()
                pgather(c + 1)

            @pl.when(jnp.logical_and(c > 3, even))
            def _():
                out_handle(c - 4, osem0).wait()

            @pl.when(jnp.logical_and(c > 3, jnp.logical_not(even)))
            def _():
                out_handle(c - 4, osem1).wait()

            @pl.when(c + 2 < n_chunks)
            def _():
                idx_handle(c + 2).wait()
                wg_handle(c + 2).start()

            @pl.when(c + 3 < n_chunks)
            def _():
                idx_handle(c + 3).start()

            means = []
            invs = []

            def group_body(gi, rcarry):
                means.clear()
                invs.clear()
                for r16 in range(LANES):
                    r = gi * LANES + r16
                    xs = [gbuf_v[s, r, pl.ds(j * LANES, LANES)]
                          for j in range(NJ)]
                    sums = xs
                    sqs = [x * x for x in xs]
                    while len(sums) > 1:
                        sums = [a + b for a, b in zip(sums[::2], sums[1::2])]
                        sqs = [a + b for a, b in zip(sqs[::2], sqs[1::2])]
                    rs = jnp.sum(sums[0])
                    rq = jnp.sum(sqs[0])
                    mean = rs * (1.0 / H)
                    var = jnp.maximum(rq * (1.0 / H) - mean * mean, 0.0)
                    means.append(mean)
                    invs.append(_rsqrt_scalar(var + EPS))
                for j in range(NJ):
                    g = gam_v[pl.ds(j * LANES, LANES)]
                    b = bet_v[pl.ds(j * LANES, LANES)]
                    for r16 in range(LANES):
                        r = gi * LANES + r16
                        x = gbuf_v[s, r, pl.ds(j * LANES, LANES)]
                        gbuf_v[s, r, pl.ds(j * LANES, LANES)] = (
                            (x - means[r16]) * invs[r16]) * g + b
                return rcarry

            lax.fori_loop(0, CHUNK // LANES, group_body, 0, unroll=True)

            @pl.when(even)
            def _():
                out_handle(c, osem0).start()

            @pl.when(jnp.logical_not(even))
            def _():
                out_handle(c, osem1).start()

            return carry

        lax.fori_loop(0, n_chunks, chunk_body, 0, unroll=False)
        out_handle(n_chunks - 4, osem0).wait()
        out_handle(n_chunks - 3, osem1).wait()
        out_handle(n_chunks - 2, osem0).wait()
        out_handle(n_chunks - 1, osem1).wait()

    return sc_call


def kernel(input_ids, position_ids, token_type_ids, word_embeddings,
           position_embeddings, token_type_embeddings, ln_gamma, ln_beta):
    b, l = input_ids.shape
    v, h = word_embeddings.shape
    p = position_embeddings.shape[0]
    t = token_type_embeddings.shape[0]
    n_rows = b * l
    ids = input_ids.reshape(n_rows).astype(jnp.int32)
    ptids = (token_type_ids.reshape(n_rows).astype(jnp.int32) * p
             + position_ids.reshape(n_rows).astype(jnp.int32))
    idx2 = jnp.stack([ids, ptids])
    ptab = (position_embeddings[None, :, :]
            + token_type_embeddings[:, None, :]).reshape(t * p, h)
    sc_call = _make_sc_call(n_rows, v, t * p)
    out = sc_call(idx2, word_embeddings, ptab, ln_gamma, ln_beta)
    return out.reshape(b, l, h)


# speedup vs baseline: 2.5776x; 1.6864x over previous
import functools

import jax
import jax.numpy as jnp
from jax import lax
from jax.experimental import pallas as pl
from jax.experimental.pallas import tpu as pltpu
from jax.experimental.pallas import tpu_sc as plsc

H = 128
LANES = 16
NJ = H // LANES
EPS = 1e-12
CHUNK = 128
NSLOT = 6


def _rsqrt_scalar(x):
    i = lax.bitcast_convert_type(x, jnp.int32)
    i = jnp.int32(0x5F3759DF) - (i >> 1)
    y = lax.bitcast_convert_type(i, jnp.float32)
    for _ in range(3):
        y = y * (1.5 - 0.5 * x * y * y)
    return y


def _make_sc_call(n_rows, v, pt_rows):
    info = plsc.get_sparse_core_info()
    nw = info.num_cores * info.num_subcores
    rows_per_w = n_rows // nw
    n_chunks = rows_per_w // CHUNK
    mesh = plsc.VectorSubcoreMesh(core_axis_name="c", subcore_axis_name="s")

    @functools.partial(
        pl.kernel,
        out_type=jax.ShapeDtypeStruct((n_rows, H), jnp.float32),
        mesh=mesh,
        scratch_types=[
            pltpu.VMEM((NSLOT, 2, CHUNK), jnp.int32),
            pltpu.VMEM((NSLOT, CHUNK, H), jnp.float32),
            pltpu.VMEM((H,), jnp.float32),
            pltpu.VMEM((H,), jnp.float32),
            pltpu.SemaphoreType.DMA,
            pltpu.SemaphoreType.DMA,
            pltpu.SemaphoreType.DMA,
            pltpu.SemaphoreType.DMA,
            pltpu.SemaphoreType.DMA,
            pltpu.SemaphoreType.DMA,
        ],
        compiler_params=pltpu.CompilerParams(needs_layout_passes=False),
    )
    def sc_call(idx2_h, word_h, ptab_h, gam_h, bet_h, out_h,
                idx_v, gbuf_v, gam_v, bet_v,
                wsem, psem0, psem1, osem0, osem1, isem):
        wid = lax.axis_index("s") * info.num_cores + lax.axis_index("c")
        base_w = wid * rows_per_w

        pltpu.sync_copy(gam_h, gam_v)
        pltpu.sync_copy(bet_h, bet_v)

        def idx_handle(c):
            return pltpu.make_async_copy(
                idx2_h.at[:, pl.ds(base_w + c * CHUNK, CHUNK)],
                idx_v.at[c % NSLOT], isem)

        def wg_handle(c):
            s = c % NSLOT
            return pltpu.make_async_copy(word_h.at[idx_v.at[s, 0]],
                                         gbuf_v.at[s], wsem)

        HC = CHUNK // 2

        def pgather(c):
            s = c % NSLOT
            pltpu.async_copy(ptab_h.at[idx_v.at[s, 1, pl.ds(0, HC)]],
                             gbuf_v.at[s, pl.ds(0, HC)], psem0, add=True)
            pltpu.async_copy(ptab_h.at[idx_v.at[s, 1, pl.ds(HC, HC)]],
                             gbuf_v.at[s, pl.ds(HC, HC)], psem1, add=True)

        def wait_pgather_lo(c):
            s = c % NSLOT
            pltpu.make_async_copy(ptab_h.at[idx_v.at[s, 1, pl.ds(0, HC)]],
                                  gbuf_v.at[s, pl.ds(0, HC)], psem0).wait()

        def wait_pgather_hi(c):
            s = c % NSLOT
            pltpu.make_async_copy(ptab_h.at[idx_v.at[s, 1, pl.ds(HC, HC)]],
                                  gbuf_v.at[s, pl.ds(HC, HC)], psem1).wait()

        def out_handle(c, sem):
            return pltpu.make_async_copy(
                gbuf_v.at[c % NSLOT],
                out_h.at[pl.ds(base_w + c * CHUNK, CHUNK)], sem)

        def out_start_half(c, half, sem):
            s = c % NSLOT
            pltpu.make_async_copy(
                gbuf_v.at[s, pl.ds(half * HC, HC)],
                out_h.at[pl.ds(base_w + c * CHUNK + half * HC, HC)],
                sem).start()

        pltpu.sync_copy(idx2_h.at[:, pl.ds(base_w, CHUNK)], idx_v.at[0])
        wg_handle(0).start()
        wg_handle(0).wait()
        pgather(0)
        pltpu.sync_copy(idx2_h.at[:, pl.ds(base_w + CHUNK, CHUNK)],
                        idx_v.at[1])
        wg_handle(1).start()
        idx_handle(2).start()

        def chunk_body(c, carry):
            s = c % NSLOT
            even = (c % 2) == 0
            wait_pgather_lo(c)

            @pl.when(c + 1 < n_chunks)
            def _():
                wg_handle(c + 1).wait()
                pgather(c + 1)

            @pl.when(jnp.logical_and(c > 3, even))
            def _():
                out_handle(c - 4, osem0).wait()

            @pl.when(jnp.logical_and(c > 3, jnp.logical_not(even)))
            def _():
                out_handle(c - 4, osem1).wait()

            @pl.when(c + 2 < n_chunks)
            def _():
                idx_handle(c + 2).wait()
                wg_handle(c + 2).start()

            @pl.when(c + 3 < n_chunks)
            def _():
                idx_handle(c + 3).start()

            means = []
            invs = []

            def group_body(gi, rcarry):
                means.clear()
                invs.clear()
                for r16 in range(LANES):
                    r = gi * LANES + r16
                    xs = [gbuf_v[s, r, pl.ds(j * LANES, LANES)]
                          for j in range(NJ)]
                    sums = xs
                    sqs = [x * x for x in xs]
                    while len(sums) > 1:
                        sums = [a + b for a, b in zip(sums[::2], sums[1::2])]
                        sqs = [a + b for a, b in zip(sqs[::2], sqs[1::2])]
                    rs = jnp.sum(sums[0])
                    rq = jnp.sum(sqs[0])
                    mean = rs * (1.0 / H)
                    var = jnp.maximum(rq * (1.0 / H) - mean * mean, 0.0)
                    means.append(mean)
                    invs.append(_rsqrt_scalar(var + EPS))
                for j in range(NJ):
                    g = gam_v[pl.ds(j * LANES, LANES)]
                    b = bet_v[pl.ds(j * LANES, LANES)]
                    for r16 in range(LANES):
                        r = gi * LANES + r16
                        x = gbuf_v[s, r, pl.ds(j * LANES, LANES)]
                        gbuf_v[s, r, pl.ds(j * LANES, LANES)] = (
                            (x - means[r16]) * invs[r16]) * g + b
                return rcarry

            lax.fori_loop(0, CHUNK // LANES // 2, group_body, 0,
                          unroll=False)

            @pl.when(even)
            def _():
                out_start_half(c, 0, osem0)

            @pl.when(jnp.logical_not(even))
            def _():
                out_start_half(c, 0, osem1)

            wait_pgather_hi(c)
            lax.fori_loop(CHUNK // LANES // 2, CHUNK // LANES, group_body,
                          0, unroll=False)

            @pl.when(even)
            def _():
                out_start_half(c, 1, osem0)

            @pl.when(jnp.logical_not(even))
            def _():
                out_start_half(c, 1, osem1)

            return carry

        lax.fori_loop(0, n_chunks, chunk_body, 0, unroll=False)
        out_handle(n_chunks - 4, osem0).wait()
        out_handle(n_chunks - 3, osem1).wait()
        out_handle(n_chunks - 2, osem0).wait()
        out_handle(n_chunks - 1, osem1).wait()

    return sc_call


def kernel(input_ids, position_ids, token_type_ids, word_embeddings,
           position_embeddings, token_type_embeddings, ln_gamma, ln_beta):
    b, l = input_ids.shape
    v, h = word_embeddings.shape
    p = position_embeddings.shape[0]
    t = token_type_embeddings.shape[0]
    n_rows = b * l
    ids = input_ids.reshape(n_rows).astype(jnp.int32)
    ptids = (token_type_ids.reshape(n_rows).astype(jnp.int32) * p
             + position_ids.reshape(n_rows).astype(jnp.int32))
    idx2 = jnp.stack([ids, ptids])
    ptab = (position_embeddings[None, :, :]
            + token_type_embeddings[:, None, :]).reshape(t * p, h)
    sc_call = _make_sc_call(n_rows, v, t * p)
    out = sc_call(idx2, word_embeddings, ptab, ln_gamma, ln_beta)
    return out.reshape(b, l, h)


# speedup vs baseline: 2.5821x; 1.0017x over previous
import functools

import jax
import jax.numpy as jnp
from jax import lax
from jax.experimental import pallas as pl
from jax.experimental.pallas import tpu as pltpu
from jax.experimental.pallas import tpu_sc as plsc

H = 128
LANES = 16
NJ = H // LANES
EPS = 1e-12
CHUNK = 128
NSLOT = 6


def _rsqrt_scalar(x):
    i = lax.bitcast_convert_type(x, jnp.int32)
    i = jnp.int32(0x5F3759DF) - (i >> 1)
    y = lax.bitcast_convert_type(i, jnp.float32)
    for _ in range(3):
        y = y * (1.5 - 0.5 * x * y * y)
    return y


def _make_sc_call(n_rows, v, pt_rows):
    info = plsc.get_sparse_core_info()
    nw = info.num_cores * info.num_subcores
    rows_per_w = n_rows // nw
    n_chunks = rows_per_w // CHUNK
    mesh = plsc.VectorSubcoreMesh(core_axis_name="c", subcore_axis_name="s")

    @functools.partial(
        pl.kernel,
        out_type=jax.ShapeDtypeStruct((n_rows, H), jnp.float32),
        mesh=mesh,
        scratch_types=[
            pltpu.VMEM((NSLOT, 2, CHUNK), jnp.int32),
            pltpu.VMEM((NSLOT, CHUNK, H), jnp.float32),
            pltpu.VMEM((H,), jnp.float32),
            pltpu.VMEM((H,), jnp.float32),
            pltpu.SemaphoreType.DMA,
            pltpu.SemaphoreType.DMA,
            pltpu.SemaphoreType.DMA,
            pltpu.SemaphoreType.DMA,
            pltpu.SemaphoreType.DMA,
        ],
        compiler_params=pltpu.CompilerParams(needs_layout_passes=False),
    )
    def sc_call(idx2_h, word_h, ptab_h, gam_h, bet_h, out_h,
                idx_v, gbuf_v, gam_v, bet_v,
                wsem, psem, osem0, osem1, isem):
        wid = lax.axis_index("s") * info.num_cores + lax.axis_index("c")
        base_w = wid * rows_per_w

        pltpu.sync_copy(gam_h, gam_v)
        pltpu.sync_copy(bet_h, bet_v)

        def idx_handle(c):
            return pltpu.make_async_copy(
                idx2_h.at[:, pl.ds(base_w + c * CHUNK, CHUNK)],
                idx_v.at[c % NSLOT], isem)

        def wg_handle(c):
            s = c % NSLOT
            return pltpu.make_async_copy(word_h.at[idx_v.at[s, 0]],
                                         gbuf_v.at[s], wsem)

        def pgather(c):
            s = c % NSLOT
            pltpu.async_copy(ptab_h.at[idx_v.at[s, 1]], gbuf_v.at[s],
                             psem, add=True)

        def wait_pgather(c):
            s = c % NSLOT
            pltpu.make_async_copy(ptab_h.at[idx_v.at[s, 1]], gbuf_v.at[s],
                                  psem).wait()

        def out_handle(c, sem):
            return pltpu.make_async_copy(
                gbuf_v.at[c % NSLOT],
                out_h.at[pl.ds(base_w + c * CHUNK, CHUNK)], sem)

        pltpu.sync_copy(idx2_h.at[:, pl.ds(base_w, CHUNK)], idx_v.at[0])
        wg_handle(0).start()
        wg_handle(0).wait()
        pgather(0)
        pltpu.sync_copy(idx2_h.at[:, pl.ds(base_w + CHUNK, CHUNK)],
                        idx_v.at[1])
        wg_handle(1).start()
        idx_handle(2).start()

        def chunk_body(c, carry):
            s = c % NSLOT
            even = (c % 2) == 0
            wait_pgather(c)

            @pl.when(c + 1 < n_chunks)
            def _():
                wg_handle(c + 1).wait()
                pgather(c + 1)

            @pl.when(jnp.logical_and(c > 3, even))
            def _():
                out_handle(c - 4, osem0).wait()

            @pl.when(jnp.logical_and(c > 3, jnp.logical_not(even)))
            def _():
                out_handle(c - 4, osem1).wait()

            @pl.when(c + 2 < n_chunks)
            def _():
                idx_handle(c + 2).wait()
                wg_handle(c + 2).start()

            @pl.when(c + 3 < n_chunks)
            def _():
                idx_handle(c + 3).start()

            means = []
            invs = []

            def group_body(gi, rcarry):
                means.clear()
                invs.clear()
                for r16 in range(LANES):
                    r = gi * LANES + r16
                    xs = [gbuf_v[s, r, pl.ds(j * LANES, LANES)]
                          for j in range(NJ)]
                    sums = xs
                    sqs = [x * x for x in xs]
                    while len(sums) > 1:
                        sums = [a + b for a, b in zip(sums[::2], sums[1::2])]
                        sqs = [a + b for a, b in zip(sqs[::2], sqs[1::2])]
                    rs = jnp.sum(sums[0])
                    rq = jnp.sum(sqs[0])
                    mean = rs * (1.0 / H)
                    var = jnp.maximum(rq * (1.0 / H) - mean * mean, 0.0)
                    means.append(mean)
                    invs.append(_rsqrt_scalar(var + EPS))
                for j in range(NJ):
                    g = gam_v[pl.ds(j * LANES, LANES)]
                    b = bet_v[pl.ds(j * LANES, LANES)]
                    for r16 in range(LANES):
                        r = gi * LANES + r16
                        x = gbuf_v[s, r, pl.ds(j * LANES, LANES)]
                        gbuf_v[s, r, pl.ds(j * LANES, LANES)] = (
                            (x - means[r16]) * invs[r16]) * g + b
                return rcarry

            lax.fori_loop(0, CHUNK // LANES, group_body, 0, unroll=False)

            @pl.when(even)
            def _():
                out_handle(c, osem0).start()

            @pl.when(jnp.logical_not(even))
            def _():
                out_handle(c, osem1).start()

            return carry

        lax.fori_loop(0, n_chunks, chunk_body, 0, unroll=False)
        out_handle(n_chunks - 4, osem0).wait()
        out_handle(n_chunks - 3, osem1).wait()
        out_handle(n_chunks - 2, osem0).wait()
        out_handle(n_chunks - 1, osem1).wait()

    return sc_call


def kernel(input_ids, position_ids, token_type_ids, word_embeddings,
           position_embeddings, token_type_embeddings, ln_gamma, ln_beta):
    b, l = input_ids.shape
    v, h = word_embeddings.shape
    p = position_embeddings.shape[0]
    t = token_type_embeddings.shape[0]
    n_rows = b * l
    ids = input_ids.reshape(n_rows).astype(jnp.int32)
    ptids = (token_type_ids.reshape(n_rows).astype(jnp.int32) * p
             + position_ids.reshape(n_rows).astype(jnp.int32))
    idx2 = jnp.stack([ids, ptids])
    ptab = (position_embeddings[None, :, :]
            + token_type_embeddings[:, None, :]).reshape(t * p, h)
    sc_call = _make_sc_call(n_rows, v, t * p)
    out = sc_call(idx2, word_embeddings, ptab, ln_gamma, ln_beta)
    return out.reshape(b, l, h)


# speedup vs baseline: 2.5847x; 1.0010x over previous
import functools

import jax
import jax.numpy as jnp
from jax import lax
from jax.experimental import pallas as pl
from jax.experimental.pallas import tpu as pltpu
from jax.experimental.pallas import tpu_sc as plsc

H = 128
LANES = 16
NJ = H // LANES
EPS = 1e-12
CHUNK = 128
NSLOT = 6


def _rsqrt_scalar(x):
    i = lax.bitcast_convert_type(x, jnp.int32)
    i = jnp.int32(0x5F3759DF) - (i >> 1)
    y = lax.bitcast_convert_type(i, jnp.float32)
    for _ in range(3):
        y = y * (1.5 - 0.5 * x * y * y)
    return y


def _make_sc_call(n_rows, v, pt_rows):
    info = plsc.get_sparse_core_info()
    nw = info.num_cores * info.num_subcores
    rows_per_w = n_rows // nw
    n_chunks = rows_per_w // CHUNK
    mesh = plsc.VectorSubcoreMesh(core_axis_name="c", subcore_axis_name="s")

    @functools.partial(
        pl.kernel,
        out_type=jax.ShapeDtypeStruct((n_rows, H), jnp.float32),
        mesh=mesh,
        scratch_types=[
            pltpu.VMEM((2, rows_per_w), jnp.int32),
            pltpu.VMEM((NSLOT, CHUNK, H), jnp.float32),
            pltpu.VMEM((H,), jnp.float32),
            pltpu.VMEM((H,), jnp.float32),
            pltpu.SemaphoreType.DMA,
            pltpu.SemaphoreType.DMA,
            pltpu.SemaphoreType.DMA,
            pltpu.SemaphoreType.DMA,
        ],
        compiler_params=pltpu.CompilerParams(needs_layout_passes=False),
    )
    def sc_call(idx2_h, word_h, ptab_h, gam_h, bet_h, out_h,
                idx_v, gbuf_v, gam_v, bet_v,
                wsem, psem, osem0, osem1):
        wid = lax.axis_index("s") * info.num_cores + lax.axis_index("c")
        base_w = wid * rows_per_w

        pltpu.sync_copy(gam_h, gam_v)
        pltpu.sync_copy(bet_h, bet_v)
        pltpu.sync_copy(idx2_h.at[:, pl.ds(base_w, rows_per_w)], idx_v)

        def wg_handle(c):
            return pltpu.make_async_copy(
                word_h.at[idx_v.at[0, pl.ds(c * CHUNK, CHUNK)]],
                gbuf_v.at[c % NSLOT], wsem)

        def pgather(c):
            pltpu.async_copy(
                ptab_h.at[idx_v.at[1, pl.ds(c * CHUNK, CHUNK)]],
                gbuf_v.at[c % NSLOT], psem, add=True)

        def wait_pgather(c):
            pltpu.make_async_copy(
                ptab_h.at[idx_v.at[1, pl.ds(c * CHUNK, CHUNK)]],
                gbuf_v.at[c % NSLOT], psem).wait()

        def out_handle(c, sem):
            return pltpu.make_async_copy(
                gbuf_v.at[c % NSLOT],
                out_h.at[pl.ds(base_w + c * CHUNK, CHUNK)], sem)

        wg_handle(0).start()
        wg_handle(0).wait()
        pgather(0)
        wg_handle(1).start()

        def chunk_body(c, carry):
            s = c % NSLOT
            even = (c % 2) == 0
            wait_pgather(c)

            @pl.when(c + 1 < n_chunks)
            def _():
                wg_handle(c + 1).wait()
                pgather(c + 1)

            @pl.when(jnp.logical_and(c > 3, even))
            def _():
                out_handle(c - 4, osem0).wait()

            @pl.when(jnp.logical_and(c > 3, jnp.logical_not(even)))
            def _():
                out_handle(c - 4, osem1).wait()

            @pl.when(c + 2 < n_chunks)
            def _():
                wg_handle(c + 2).start()

            means = []
            invs = []

            def group_body(gi, rcarry):
                means.clear()
                invs.clear()
                for r16 in range(LANES):
                    r = gi * LANES + r16
                    xs = [gbuf_v[s, r, pl.ds(j * LANES, LANES)]
                          for j in range(NJ)]
                    sums = xs
                    sqs = [x * x for x in xs]
                    while len(sums) > 1:
                        sums = [a + b for a, b in zip(sums[::2], sums[1::2])]
                        sqs = [a + b for a, b in zip(sqs[::2], sqs[1::2])]
                    rs = jnp.sum(sums[0])
                    rq = jnp.sum(sqs[0])
                    mean = rs * (1.0 / H)
                    var = jnp.maximum(rq * (1.0 / H) - mean * mean, 0.0)
                    means.append(mean)
                    invs.append(_rsqrt_scalar(var + EPS))
                for j in range(NJ):
                    g = gam_v[pl.ds(j * LANES, LANES)]
                    b = bet_v[pl.ds(j * LANES, LANES)]
                    for r16 in range(LANES):
                        r = gi * LANES + r16
                        x = gbuf_v[s, r, pl.ds(j * LANES, LANES)]
                        gbuf_v[s, r, pl.ds(j * LANES, LANES)] = (
                            (x - means[r16]) * invs[r16]) * g + b
                return rcarry

            lax.fori_loop(0, CHUNK // LANES, group_body, 0, unroll=False)

            @pl.when(even)
            def _():
                out_handle(c, osem0).start()

            @pl.when(jnp.logical_not(even))
            def _():
                out_handle(c, osem1).start()

            return carry

        lax.fori_loop(0, n_chunks, chunk_body, 0, unroll=False)
        out_handle(n_chunks - 4, osem0).wait()
        out_handle(n_chunks - 3, osem1).wait()
        out_handle(n_chunks - 2, osem0).wait()
        out_handle(n_chunks - 1, osem1).wait()

    return sc_call


def kernel(input_ids, position_ids, token_type_ids, word_embeddings,
           position_embeddings, token_type_embeddings, ln_gamma, ln_beta):
    b, l = input_ids.shape
    v, h = word_embeddings.shape
    p = position_embeddings.shape[0]
    t = token_type_embeddings.shape[0]
    n_rows = b * l
    ids = input_ids.reshape(n_rows).astype(jnp.int32)
    ptids = (token_type_ids.reshape(n_rows).astype(jnp.int32) * p
             + position_ids.reshape(n_rows).astype(jnp.int32))
    idx2 = jnp.stack([ids, ptids])
    ptab = (position_embeddings[None, :, :]
            + token_type_embeddings[:, None, :]).reshape(t * p, h)
    sc_call = _make_sc_call(n_rows, v, t * p)
    out = sc_call(idx2, word_embeddings, ptab, ln_gamma, ln_beta)
    return out.reshape(b, l, h)
